# baseline (speedup 1.0000x reference)
import jax
import jax.numpy as jnp
from jax.experimental import pallas as pl
from jax.experimental.pallas import tpu as pltpu

_N = 20000
_NPAD = 20480
_ROWS = 160
_K = 1000
_MAX_DETS = 100
_THR = 0.5
_NEG = float("-inf")


def _nms_body(x1_ref, y1_ref, x2_ref, y2_ref, s_ref, out_ref, idx_ref, ab_ref):
    S = s_ref[...]
    bits = jax.lax.bitcast_convert_type(S, jnp.int32)
    IDX = (jax.lax.broadcasted_iota(jnp.int32, (_ROWS, 128), 0) * 128
           + jax.lax.broadcasted_iota(jnp.int32, (_ROWS, 128), 1))
    idx_ref[...] = IDX

    def bs1(_, lohi):
        lo, hi = lohi
        mid = lo + (hi - lo) // 2
        cnt = jnp.sum((bits >= mid).astype(jnp.int32))
        ge = cnt >= _K
        return (jnp.where(ge, mid, lo), jnp.where(ge, hi, mid))

    lo, _ = jax.lax.fori_loop(0, 31, bs1, (jnp.int32(0), jnp.int32(0x7F800000)))
    n1 = jnp.sum((bits > lo).astype(jnp.int32))
    m = _K - n1
    tie = bits == lo

    def bs2(_, lohi):
        lo2, hi2 = lohi
        mid = (lo2 + hi2) // 2
        cnt = jnp.sum((tie & (IDX < mid)).astype(jnp.int32))
        ge = cnt >= m
        return (jnp.where(ge, lo2, mid), jnp.where(ge, mid, hi2))

    _, p = jax.lax.fori_loop(0, 15, bs2, (jnp.int32(0), jnp.int32(_NPAD)))
    sel = (bits > lo) | (tie & (IDX < p))
    s0 = jnp.where(sel, S, _NEG)

    ab_ref[...] = (x2_ref[...] - x1_ref[...]) * (y2_ref[...] - y1_ref[...])

    def fold(av, ai, bv, bi):
        ta = (av > bv) | ((av == bv) & (ai < bi))
        return jnp.where(ta, av, bv), jnp.where(ta, ai, bi)

    def argmax_joint(v, ix):
        for h in (80, 40, 20, 10, 5):
            v, ix = fold(v[:h], ix[:h], v[h:], ix[h:])
        v2, i2 = fold(v[0:2], ix[0:2], v[2:4], ix[2:4])
        v1, i1 = fold(v2[0:1], i2[0:1], v2[1:2], i2[1:2])
        v1, i1 = fold(v1, i1, v[4:5], ix[4:5])
        for w in (64, 32, 16, 8, 4, 2, 1):
            v1, i1 = fold(v1[:, :w], i1[:, :w], v1[:, w:2 * w], i1[:, w:2 * w])
        return v1[0, 0], i1[0, 0]

    def nms(i, carry):
        Sv, out = carry
        IDXv = idx_ref[...]
        M, idx = argmax_joint(Sv, IDXv)
        valid = M > _NEG
        r = idx // 128
        c = idx % 128
        lm = jax.lax.broadcasted_iota(jnp.int32, (1, 128), 1) == c
        bx1 = jnp.sum(jnp.where(lm, x1_ref[pl.ds(r, 1), :], 0.0))
        by1 = jnp.sum(jnp.where(lm, y1_ref[pl.ds(r, 1), :], 0.0))
        bx2 = jnp.sum(jnp.where(lm, x2_ref[pl.ds(r, 1), :], 0.0))
        by2 = jnp.sum(jnp.where(lm, y2_ref[pl.ds(r, 1), :], 0.0))
        xx1 = jnp.maximum(bx1, x1_ref[...])
        yy1 = jnp.maximum(by1, y1_ref[...])
        xx2 = jnp.minimum(bx2, x2_ref[...])
        yy2 = jnp.minimum(by2, y2_ref[...])
        inter = jnp.maximum(xx2 - xx1, 0.0) * jnp.maximum(yy2 - yy1, 0.0)
        area_a = (bx2 - bx1) * (by2 - by1)
        union = area_a + ab_ref[...] - inter
        iou = inter / jnp.maximum(union, 1e-9)
        new_s = jnp.where((iou >= _THR) | (IDXv == idx), _NEG, Sv)

        row = jax.lax.broadcasted_iota(jnp.int32, (128, 8), 0)
        lane = jax.lax.broadcasted_iota(jnp.int32, (128, 8), 1)
        z = jnp.float32(0.0)
        vals = (jnp.where(lane == 0, jnp.where(valid, bx1, z), z)
                + jnp.where(lane == 1, jnp.where(valid, by1, z), z)
                + jnp.where(lane == 2, jnp.where(valid, bx2, z), z)
                + jnp.where(lane == 3, jnp.where(valid, by2, z), z)
                + jnp.where(lane == 4, jnp.where(valid, M, z), z))
        return new_s, jnp.where(row == i, vals, out)

    _, outv = jax.lax.fori_loop(
        0, _MAX_DETS, nms, (s0, jnp.zeros((128, 8), jnp.float32)))
    out_ref[...] = outv


def kernel(boxes, scores):
    b = jnp.pad(boxes, ((0, _NPAD - _N), (0, 0)))
    s = jnp.pad(scores, (0, _NPAD - _N), constant_values=-1.0)
    x1 = b[:, 0].reshape(_ROWS, 128)
    y1 = b[:, 1].reshape(_ROWS, 128)
    x2 = b[:, 2].reshape(_ROWS, 128)
    y2 = b[:, 3].reshape(_ROWS, 128)
    out = pl.pallas_call(
        _nms_body,
        out_shape=jax.ShapeDtypeStruct((128, 8), jnp.float32),
        scratch_shapes=[
            pltpu.VMEM((_ROWS, 128), jnp.int32),
            pltpu.VMEM((_ROWS, 128), jnp.float32),
        ],
    )(x1, y1, x2, y2, s.reshape(_ROWS, 128))
    return out[:_MAX_DETS, :5]


# speedup vs baseline: 1.2997x; 1.2997x over previous
import jax
import jax.numpy as jnp
from jax.experimental import pallas as pl
from jax.experimental.pallas import tpu as pltpu

_N = 20000
_NPAD = 20480
_ROWS = 160
_K = 1000
_MAX_DETS = 100
_THR = 0.5
_NEG = float("-inf")


def _nms_body(x1_ref, y1_ref, x2_ref, y2_ref, s_ref, out_ref, idx_ref, ab_ref):
    S = s_ref[...]
    bits = jax.lax.bitcast_convert_type(S, jnp.int32)
    IDX = (jax.lax.broadcasted_iota(jnp.int32, (_ROWS, 128), 0) * 128
           + jax.lax.broadcasted_iota(jnp.int32, (_ROWS, 128), 1))
    idx_ref[...] = IDX

    def bs1(_, lohi):
        lo, hi = lohi
        mid = lo + (hi - lo) // 2
        cnt = jnp.sum((bits >= mid).astype(jnp.int32))
        ge = cnt >= _K
        return (jnp.where(ge, mid, lo), jnp.where(ge, hi, mid))

    lo, _ = jax.lax.fori_loop(0, 31, bs1, (jnp.int32(0), jnp.int32(0x7F800000)))
    n1 = jnp.sum((bits > lo).astype(jnp.int32))
    m = _K - n1
    tie = bits == lo

    def bs2(_, lohi):
        lo2, hi2 = lohi
        mid = (lo2 + hi2) // 2
        cnt = jnp.sum((tie & (IDX < mid)).astype(jnp.int32))
        ge = cnt >= m
        return (jnp.where(ge, lo2, mid), jnp.where(ge, mid, hi2))

    _, p = jax.lax.fori_loop(0, 15, bs2, (jnp.int32(0), jnp.int32(_NPAD)))
    sel = (bits > lo) | (tie & (IDX < p))
    s0 = jnp.where(sel, S, _NEG)

    ab_ref[...] = (x2_ref[...] - x1_ref[...]) * (y2_ref[...] - y1_ref[...])

    def nms(i, carry):
        Sv, out = carry
        IDXv = idx_ref[...]
        M = jnp.max(Sv)
        valid = M > _NEG
        idx = jnp.argmax(Sv.reshape(-1)).astype(jnp.int32)
        r = idx // 128
        c = idx % 128
        lm = jax.lax.broadcasted_iota(jnp.int32, (1, 128), 1) == c
        bx1 = jnp.sum(jnp.where(lm, x1_ref[pl.ds(r, 1), :], 0.0))
        by1 = jnp.sum(jnp.where(lm, y1_ref[pl.ds(r, 1), :], 0.0))
        bx2 = jnp.sum(jnp.where(lm, x2_ref[pl.ds(r, 1), :], 0.0))
        by2 = jnp.sum(jnp.where(lm, y2_ref[pl.ds(r, 1), :], 0.0))
        xx1 = jnp.maximum(bx1, x1_ref[...])
        yy1 = jnp.maximum(by1, y1_ref[...])
        xx2 = jnp.minimum(bx2, x2_ref[...])
        yy2 = jnp.minimum(by2, y2_ref[...])
        inter = jnp.maximum(xx2 - xx1, 0.0) * jnp.maximum(yy2 - yy1, 0.0)
        area_a = (bx2 - bx1) * (by2 - by1)
        union = area_a + ab_ref[...] - inter
        iou = inter / jnp.maximum(union, 1e-9)
        new_s = jnp.where((iou >= _THR) | (IDXv == idx), _NEG, Sv)

        row = jax.lax.broadcasted_iota(jnp.int32, (128, 8), 0)
        lane = jax.lax.broadcasted_iota(jnp.int32, (128, 8), 1)
        z = jnp.float32(0.0)
        vals = (jnp.where(lane == 0, jnp.where(valid, bx1, z), z)
                + jnp.where(lane == 1, jnp.where(valid, by1, z), z)
                + jnp.where(lane == 2, jnp.where(valid, bx2, z), z)
                + jnp.where(lane == 3, jnp.where(valid, by2, z), z)
                + jnp.where(lane == 4, jnp.where(valid, M, z), z))
        return new_s, jnp.where(row == i, vals, out)

    _, outv = jax.lax.fori_loop(
        0, _MAX_DETS, nms, (s0, jnp.zeros((128, 8), jnp.float32)))
    out_ref[...] = outv


def kernel(boxes, scores):
    b = jnp.pad(boxes, ((0, _NPAD - _N), (0, 0)))
    s = jnp.pad(scores, (0, _NPAD - _N), constant_values=-1.0)
    x1 = b[:, 0].reshape(_ROWS, 128)
    y1 = b[:, 1].reshape(_ROWS, 128)
    x2 = b[:, 2].reshape(_ROWS, 128)
    y2 = b[:, 3].reshape(_ROWS, 128)
    out = pl.pallas_call(
        _nms_body,
        out_shape=jax.ShapeDtypeStruct((128, 8), jnp.float32),
        scratch_shapes=[
            pltpu.VMEM((_ROWS, 128), jnp.int32),
            pltpu.VMEM((_ROWS, 128), jnp.float32),
        ],
    )(x1, y1, x2, y2, s.reshape(_ROWS, 128))
    return out[:_MAX_DETS, :5]


# speedup vs baseline: 1.4005x; 1.0776x over previous
import jax
import jax.numpy as jnp
from jax.experimental import pallas as pl
from jax.experimental.pallas import tpu as pltpu

_N = 20000
_NPAD = 20480
_ROWS = 160
_K = 1000
_MAX_DETS = 100
_THR = 0.5
_NEG = float("-inf")


def _nms_body(x1_ref, y1_ref, x2_ref, y2_ref, s_ref, out_ref, idx_ref, ab_ref):
    S = s_ref[...]
    bits = jax.lax.bitcast_convert_type(S, jnp.int32)
    IDX = (jax.lax.broadcasted_iota(jnp.int32, (_ROWS, 128), 0) * 128
           + jax.lax.broadcasted_iota(jnp.int32, (_ROWS, 128), 1))
    idx_ref[...] = IDX

    def bs1(_, lohi):
        lo, hi = lohi
        mid = lo + (hi - lo) // 2
        cnt = jnp.sum((bits >= mid).astype(jnp.int32))
        ge = cnt >= _K
        return (jnp.where(ge, mid, lo), jnp.where(ge, hi, mid))

    lo, _ = jax.lax.fori_loop(0, 31, bs1, (jnp.int32(0), jnp.int32(0x7F800000)))
    n1 = jnp.sum((bits > lo).astype(jnp.int32))
    m = _K - n1
    tie = bits == lo

    def bs2(_, lohi):
        lo2, hi2 = lohi
        mid = (lo2 + hi2) // 2
        cnt = jnp.sum((tie & (IDX < mid)).astype(jnp.int32))
        ge = cnt >= m
        return (jnp.where(ge, lo2, mid), jnp.where(ge, mid, hi2))

    _, p = jax.lax.fori_loop(0, 15, bs2, (jnp.int32(0), jnp.int32(_NPAD)))
    sel = (bits > lo) | (tie & (IDX < p))
    s0 = jnp.where(sel, S, _NEG)

    ab_ref[...] = (x2_ref[...] - x1_ref[...]) * (y2_ref[...] - y1_ref[...])

    def nms(i, carry):
        Sv, out = carry
        IDXv = idx_ref[...]
        M = jnp.max(Sv)
        valid = M > _NEG
        idx = jnp.min(jnp.where(Sv == M, IDXv, jnp.int32(0x7FFFFFFF)))
        r = idx // 128
        c = idx % 128
        lm = jax.lax.broadcasted_iota(jnp.int32, (1, 128), 1) == c
        bx1 = jnp.sum(jnp.where(lm, x1_ref[pl.ds(r, 1), :], 0.0))
        by1 = jnp.sum(jnp.where(lm, y1_ref[pl.ds(r, 1), :], 0.0))
        bx2 = jnp.sum(jnp.where(lm, x2_ref[pl.ds(r, 1), :], 0.0))
        by2 = jnp.sum(jnp.where(lm, y2_ref[pl.ds(r, 1), :], 0.0))
        xx1 = jnp.maximum(bx1, x1_ref[...])
        yy1 = jnp.maximum(by1, y1_ref[...])
        xx2 = jnp.minimum(bx2, x2_ref[...])
        yy2 = jnp.minimum(by2, y2_ref[...])
        inter = jnp.maximum(xx2 - xx1, 0.0) * jnp.maximum(yy2 - yy1, 0.0)
        area_a = (bx2 - bx1) * (by2 - by1)
        union = area_a + ab_ref[...] - inter
        iou = inter / jnp.maximum(union, 1e-9)
        new_s = jnp.where((iou >= _THR) | (IDXv == idx), _NEG, Sv)

        row = jax.lax.broadcasted_iota(jnp.int32, (8, 128), 0)
        col = jax.lax.broadcasted_iota(jnp.int32, (8, 128), 1)
        z = jnp.float32(0.0)
        vals = (jnp.where(row == 0, jnp.where(valid, bx1, z), z)
                + jnp.where(row == 1, jnp.where(valid, by1, z), z)
                + jnp.where(row == 2, jnp.where(valid, bx2, z), z)
                + jnp.where(row == 3, jnp.where(valid, by2, z), z)
                + jnp.where(row == 4, jnp.where(valid, M, z), z))
        return new_s, jnp.where(col == i, vals, out)

    _, outv = jax.lax.fori_loop(
        0, _MAX_DETS, nms, (s0, jnp.zeros((8, 128), jnp.float32)))
    out_ref[...] = outv


def kernel(boxes, scores):
    b = jnp.pad(boxes, ((0, _NPAD - _N), (0, 0)))
    s = jnp.pad(scores, (0, _NPAD - _N), constant_values=-1.0)
    x1 = b[:, 0].reshape(_ROWS, 128)
    y1 = b[:, 1].reshape(_ROWS, 128)
    x2 = b[:, 2].reshape(_ROWS, 128)
    y2 = b[:, 3].reshape(_ROWS, 128)
    out = pl.pallas_call(
        _nms_body,
        out_shape=jax.ShapeDtypeStruct((8, 128), jnp.float32),
        scratch_shapes=[
            pltpu.VMEM((_ROWS, 128), jnp.int32),
            pltpu.VMEM((_ROWS, 128), jnp.float32),
        ],
    )(x1, y1, x2, y2, s.reshape(_ROWS, 128))
    return out[:5, :_MAX_DETS].T


# speedup vs baseline: 1.4597x; 1.0423x over previous
import jax
import jax.numpy as jnp
from jax.experimental import pallas as pl
from jax.experimental.pallas import tpu as pltpu

_N = 20000
_NPAD = 20480
_ROWS = 160
_K = 1000
_MAX_DETS = 100
_THR = 0.5
_NEG = float("-inf")


def _nms_body(x1_ref, y1_ref, x2_ref, y2_ref, s_ref, out_ref, idx_ref, ab_ref):
    S = s_ref[...]
    bits = jax.lax.bitcast_convert_type(S, jnp.int32)
    IDX = (jax.lax.broadcasted_iota(jnp.int32, (_ROWS, 128), 0) * 128
           + jax.lax.broadcasted_iota(jnp.int32, (_ROWS, 128), 1))
    idx_ref[...] = IDX

    def bs1(_, lohi):
        lo, hi = lohi
        d = jnp.maximum((hi - lo) // 4, 1)
        m1 = lo + d
        m2 = lo + 2 * d
        m3 = lo + 3 * d
        g1 = jnp.sum((bits >= m1).astype(jnp.int32)) >= _K
        g2 = jnp.sum((bits >= m2).astype(jnp.int32)) >= _K
        g3 = jnp.sum((bits >= m3).astype(jnp.int32)) >= _K
        nlo = jnp.where(g3, m3, jnp.where(g2, m2, jnp.where(g1, m1, lo)))
        nhi = jnp.where(g3, hi, jnp.where(g2, m3, jnp.where(g1, m2, m1)))
        return nlo, nhi

    lo, _ = jax.lax.fori_loop(0, 16, bs1, (jnp.int32(0), jnp.int32(0x7F800000)))
    n1 = jnp.sum((bits > lo).astype(jnp.int32))
    m = _K - n1
    tie = bits == lo

    def bs2(_, lohi):
        lo2, hi2 = lohi
        d = jnp.maximum((hi2 - lo2) // 4, 1)
        m1 = lo2 + d
        m2 = lo2 + 2 * d
        m3 = lo2 + 3 * d
        g1 = jnp.sum((tie & (IDX < m1)).astype(jnp.int32)) >= m
        g2 = jnp.sum((tie & (IDX < m2)).astype(jnp.int32)) >= m
        g3 = jnp.sum((tie & (IDX < m3)).astype(jnp.int32)) >= m
        nhi = jnp.where(g1, m1, jnp.where(g2, m2, jnp.where(g3, m3, hi2)))
        nlo = jnp.where(g1, lo2, jnp.where(g2, m1, jnp.where(g3, m2, m3)))
        return nlo, nhi

    _, p = jax.lax.fori_loop(0, 8, bs2, (jnp.int32(0), jnp.int32(_NPAD)))
    sel = (bits > lo) | (tie & (IDX < p))
    s0 = jnp.where(sel, S, _NEG)

    ab_ref[...] = (x2_ref[...] - x1_ref[...]) * (y2_ref[...] - y1_ref[...])

    def nms(i, carry):
        Sv, out = carry
        IDXv = idx_ref[...]
        M = jnp.max(Sv)
        valid = M > _NEG
        idx = jnp.min(jnp.where(Sv == M, IDXv, jnp.int32(0x7FFFFFFF)))
        r = idx // 128
        c = idx % 128
        lm = jax.lax.broadcasted_iota(jnp.int32, (1, 128), 1) == c
        bx1 = jnp.sum(jnp.where(lm, x1_ref[pl.ds(r, 1), :], 0.0))
        by1 = jnp.sum(jnp.where(lm, y1_ref[pl.ds(r, 1), :], 0.0))
        bx2 = jnp.sum(jnp.where(lm, x2_ref[pl.ds(r, 1), :], 0.0))
        by2 = jnp.sum(jnp.where(lm, y2_ref[pl.ds(r, 1), :], 0.0))
        xx1 = jnp.maximum(bx1, x1_ref[...])
        yy1 = jnp.maximum(by1, y1_ref[...])
        xx2 = jnp.minimum(bx2, x2_ref[...])
        yy2 = jnp.minimum(by2, y2_ref[...])
        inter = jnp.maximum(xx2 - xx1, 0.0) * jnp.maximum(yy2 - yy1, 0.0)
        area_a = (bx2 - bx1) * (by2 - by1)
        union = area_a + ab_ref[...] - inter
        iou = inter / jnp.maximum(union, 1e-9)
        new_s = jnp.where((iou >= _THR) | (IDXv == idx), _NEG, Sv)

        row = jax.lax.broadcasted_iota(jnp.int32, (8, 128), 0)
        col = jax.lax.broadcasted_iota(jnp.int32, (8, 128), 1)
        z = jnp.float32(0.0)
        vals = (jnp.where(row == 0, jnp.where(valid, bx1, z), z)
                + jnp.where(row == 1, jnp.where(valid, by1, z), z)
                + jnp.where(row == 2, jnp.where(valid, bx2, z), z)
                + jnp.where(row == 3, jnp.where(valid, by2, z), z)
                + jnp.where(row == 4, jnp.where(valid, M, z), z))
        return new_s, jnp.where(col == i, vals, out)

    _, outv = jax.lax.fori_loop(
        0, _MAX_DETS, nms, (s0, jnp.zeros((8, 128), jnp.float32)))
    out_ref[...] = outv


def kernel(boxes, scores):
    b = jnp.pad(boxes, ((0, _NPAD - _N), (0, 0)))
    s = jnp.pad(scores, (0, _NPAD - _N), constant_values=-1.0)
    x1 = b[:, 0].reshape(_ROWS, 128)
    y1 = b[:, 1].reshape(_ROWS, 128)
    x2 = b[:, 2].reshape(_ROWS, 128)
    y2 = b[:, 3].reshape(_ROWS, 128)
    out = pl.pallas_call(
        _nms_body,
        out_shape=jax.ShapeDtypeStruct((8, 128), jnp.float32),
        scratch_shapes=[
            pltpu.VMEM((_ROWS, 128), jnp.int32),
            pltpu.VMEM((_ROWS, 128), jnp.float32),
        ],
    )(x1, y1, x2, y2, s.reshape(_ROWS, 128))
    return out[:5, :_MAX_DETS].T
